# baseline (speedup 1.0000x reference)
import functools

import jax
import jax.numpy as jnp
from jax.experimental import pallas as pl

_E = 8
_KH = _KW = 3


def _fmt_kernel(x_ref, x1_ref, pool_ref, *, n_rb):
    i = pl.program_id(1)

    @pl.when(i == 0)
    def _():
        pool_ref[...] = jnp.zeros_like(pool_ref)

    @pl.when((i >= 1) & (i <= n_rb))
    def _():
        pool_ref[...] += jnp.sum(x_ref[...], axis=(2, 3))[:, None, :]
        t = jnp.transpose(x_ref[0].astype(jnp.bfloat16), (1, 2, 0))
        x1_ref[0, :, 0 : t.shape[1], :] = t
        x1_ref[0, :, t.shape[1] :, :] = jnp.zeros_like(
            x1_ref[0, :, t.shape[1] :, :])

    @pl.when((i == 0) | (i == n_rb + 1))
    def _():
        x1_ref[...] = jnp.zeros_like(x1_ref)


def _gate_kernel(pooled_ref, gwt_ref, gb_ref, ew_ref, eb_ref, k_ref,
                 cw_ref, cb_ref, *, n_pixels):
    pooled = pooled_ref[:, 0, :] * (1.0 / n_pixels)
    b = pooled.shape[0]
    chan = pooled.shape[1]
    logits = jnp.sum(pooled[:, None, :] * gwt_ref[...][None, :, :], axis=2)
    logits = logits + gb_ref[...]
    m = jnp.max(logits, axis=1, keepdims=True)
    ex = jnp.exp(logits - m)
    w = ex / jnp.sum(ex, axis=1, keepdims=True)

    iota = jax.lax.broadcasted_iota(jnp.int32, w.shape, 1)
    m1 = jnp.max(w, axis=1, keepdims=True)
    i1 = jnp.min(jnp.where(w == m1, iota, _E), axis=1, keepdims=True)
    w2 = jnp.where(iota == i1, -jnp.inf, w)
    m2 = jnp.max(w2, axis=1, keepdims=True)
    i2 = jnp.min(jnp.where(w2 == m2, iota, _E), axis=1, keepdims=True)
    ew = jnp.where(iota == i1, m1, 0.0) + jnp.where(iota == i2, m2, 0.0)
    scale = ew * k_ref[0, 0]

    center = _KW * (_KH // 2) + _KW // 2
    t3 = jax.lax.broadcasted_iota(jnp.int32, (_KH * _KW, chan, chan), 0)
    rr = jax.lax.broadcasted_iota(jnp.int32, (_KH * _KW, chan, chan), 1)
    cc = jax.lax.broadcasted_iota(jnp.int32, (_KH * _KW, chan, chan), 2)
    eye3 = ((t3 == center) & (rr == cc)).astype(jnp.float32)

    for bi in range(b):
        acc_w = scale[bi, 0] * ew_ref[0]
        acc_b = scale[bi, 0] * eb_ref[0:1, :]
        for e in range(1, _E):
            acc_w = acc_w + scale[bi, e] * ew_ref[e]
            acc_b = acc_b + scale[bi, e] * eb_ref[e:e + 1, :]
        cw_ref[bi] = acc_w + eye3
        cb_ref[bi] = acc_b


def _conv_kernel(x1_ref, w_ref, b_ref, out_ref, *, th, width, chan, rb):
    i = pl.program_id(1)
    row0 = i * th
    wf = width + 8
    m = th * wf
    dn = (((1,), (0,)), ((), ()))
    accs = [jnp.zeros((m, chan), jnp.float32) for _ in range(_KW)]
    for dy in range(_KH):
        slab = x1_ref[0, pl.ds(row0 + rb - 1 + dy, th), :, :]
        flat = slab.reshape(m, chan)
        for dx in range(_KW):
            wtap = w_ref[0, _KW * dy + dx, :, :].astype(jnp.bfloat16)
            accs[dx] = accs[dx] + jax.lax.dot_general(
                flat, wtap, dn, preferred_element_type=jnp.float32)
    zrow = jnp.zeros((1, chan), jnp.float32)
    out = (accs[1]
           + jnp.concatenate([zrow, accs[0][:-1, :]], axis=0)
           + jnp.concatenate([accs[2][1:, :], zrow], axis=0))
    out = out.reshape(th, wf, chan)[:, 0:width, :] + b_ref[0]
    out_ref[...] = jnp.transpose(out, (2, 0, 1))[None]


def kernel(inputs, k, expert_w, expert_b, gate_w, gate_b):
    bsz, chan, height, width = inputs.shape
    n_pixels = height * width
    rb = 32
    n_rb = height // rb

    ew9 = expert_w.transpose(0, 3, 4, 2, 1).reshape(_E, _KH * _KW, chan, chan)
    gwt = gate_w.T
    gb2 = gate_b.reshape(1, _E)
    k2 = k.reshape(1, 1)

    hp = height + 2 * rb
    wp = width + 8
    x1, pooled = pl.pallas_call(
        functools.partial(_fmt_kernel, n_rb=n_rb),
        grid=(bsz, n_rb + 2),
        in_specs=[pl.BlockSpec(
            (1, chan, rb, width),
            lambda b, i: (b, 0, jnp.clip(i - 1, 0, n_rb - 1), 0))],
        out_specs=[
            pl.BlockSpec((1, rb, wp, chan), lambda b, i: (b, i, 0, 0)),
            pl.BlockSpec((1, 1, chan), lambda b, i: (b, 0, 0)),
        ],
        out_shape=[
            jax.ShapeDtypeStruct((bsz, hp, wp, chan), jnp.bfloat16),
            jax.ShapeDtypeStruct((bsz, 1, chan), jnp.float32),
        ],
    )(inputs)

    cw, cb = pl.pallas_call(
        functools.partial(_gate_kernel, n_pixels=n_pixels),
        in_specs=[
            pl.BlockSpec(pooled.shape, lambda: (0, 0, 0)),
            pl.BlockSpec(gwt.shape, lambda: (0, 0)),
            pl.BlockSpec(gb2.shape, lambda: (0, 0)),
            pl.BlockSpec(ew9.shape, lambda: (0, 0, 0, 0)),
            pl.BlockSpec(expert_b.shape, lambda: (0, 0)),
            pl.BlockSpec(k2.shape, lambda: (0, 0)),
        ],
        out_specs=[
            pl.BlockSpec((bsz, _KH * _KW, chan, chan), lambda: (0, 0, 0, 0)),
            pl.BlockSpec((bsz, 1, chan), lambda: (0, 0, 0)),
        ],
        out_shape=[
            jax.ShapeDtypeStruct((bsz, _KH * _KW, chan, chan), jnp.float32),
            jax.ShapeDtypeStruct((bsz, 1, chan), jnp.float32),
        ],
    )(pooled, gwt, gb2, ew9, expert_b, k2)

    th = 32
    ni = height // th
    out = pl.pallas_call(
        functools.partial(_conv_kernel, th=th, width=width, chan=chan, rb=rb),
        grid=(bsz, ni),
        in_specs=[
            pl.BlockSpec((1, hp, wp, chan), lambda b, i: (b, 0, 0, 0)),
            pl.BlockSpec((1, _KH * _KW, chan, chan), lambda b, i: (b, 0, 0, 0)),
            pl.BlockSpec((1, 1, chan), lambda b, i: (b, 0, 0)),
        ],
        out_specs=pl.BlockSpec((1, chan, th, width), lambda b, i: (b, 0, i, 0)),
        out_shape=jax.ShapeDtypeStruct((bsz, chan, height, width), jnp.float32),
    )(x1, cw, cb)

    return out
